# baseline (speedup 1.0000x reference)
import functools

import jax
import jax.numpy as jnp
from jax import lax
from jax.experimental import pallas as pl
from jax.experimental.pallas import tpu as pltpu
from jax.experimental.pallas import tpu_sc as plsc

V = 65536
D = 256
B = 49152
NC, NS, L = 2, 16, 16
NW = NC * NS
BPW = B // NW
PW = V // 4
NPV = PW // L
BLK = 16
NBLK = NPV // BLK
WWIN = 18432
NWV = WWIN // L
GB = 96
NGB = BPW // GB
IBUF = BPW + L

_mesh = plsc.VectorSubcoreMesh(core_axis_name="c", subcore_axis_name="s")
_params = pltpu.CompilerParams(needs_layout_passes=False)


def _bytesum(v):
    s = v + (v >> 16)
    s = s + (s >> 8)
    return s & 0xFF


@functools.partial(
    pl.kernel,
    out_type=jax.ShapeDtypeStruct((B, D), jnp.float32),
    mesh=_mesh,
    scratch_types=[
        pltpu.VMEM((PW,), jnp.int32),
        pltpu.VMEM((NBLK, L), jnp.int32),
        pltpu.VMEM((WWIN,), jnp.int32),
        pltpu.VMEM((IBUF,), jnp.int32),
        pltpu.VMEM((2, GB, D), jnp.float32),
        pltpu.SemaphoreType.DMA,
        pltpu.SemaphoreType.DMA,
    ],
    compiler_params=_params,
)
def _compact_gather(maskp_hbm, mask_hbm, states_hbm, out_hbm, pbuf, bsums,
                    wbuf, ibuf, rowbuf, sem0, sem1):
    wid = lax.axis_index("s") * NC + lax.axis_index("c")
    lane = lax.iota(jnp.int32, L)
    target = jnp.full((L,), wid * BPW, jnp.int32)

    with jax.named_scope("mask_load"):
        pltpu.sync_copy(maskp_hbm, pbuf)

    with jax.named_scope("locate"):
        zero = jnp.zeros((L,), jnp.int32)

        def s1body(b_, _):
            acc = zero
            for t in range(BLK):
                acc = acc + _bytesum(pbuf[pl.ds(b_ * (BLK * L) + t * L, L)])
            bsums[b_, pl.ds(0, L)] = acc
            return 0

        lax.fori_loop(0, NBLK, s1body, 0)

        acc, startb, accsel = zero, zero, zero
        for s in range(NBLK):
            tot = jnp.full((L,), jnp.sum(bsums[s, pl.ds(0, L)]), jnp.int32)
            take = (acc + tot) <= target
            startb = startb + jnp.where(take, 1, 0)
            accsel = jnp.where(take, acc + tot, accsel)
            acc = acc + tot
        bstar = jnp.max(startb)

        acc2, startv, accsel2 = zero, zero, zero
        for t in range(BLK):
            v = pbuf[pl.ds(bstar * (BLK * L) + t * L, L)]
            tot = jnp.full((L,), jnp.sum(_bytesum(v)), jnp.int32)
            take = (acc2 + tot) <= (target - accsel)
            startv = startv + jnp.where(take, 1, 0)
            accsel2 = jnp.where(take, acc2 + tot, accsel2)
            acc2 = acc2 + tot
        k0 = target - accsel - accsel2
        a = (bstar * BLK + jnp.max(startv)) * (4 * L)

    with jax.named_scope("walk"):
        pltpu.sync_copy(mask_hbm.at[pl.ds(a, WWIN)], wbuf)
        coll = jnp.int32(0)
        srem = k0
        for i in range(4):
            m = wbuf[pl.ds(i * L, L)]
            ison = m > 0
            one = jnp.where(ison, 1, 0)
            pref = plsc.cumsum(one) - one
            keep = ison & (pref >= srem)
            pos = jnp.full((L,), a + i * L, jnp.int32) + lane
            plsc.store_compressed(ibuf.at[pl.ds(coll, L)], pos, mask=keep)
            coll = coll + jnp.sum(jnp.where(keep, 1, 0))
            srem = jnp.maximum(srem - plsc.all_reduce_population_count(ison),
                               0)

        def wcond(carry):
            c, vi = carry
            return (c < BPW) & (vi < NWV)

        def wbody(carry):
            c, vi = carry
            m = wbuf[pl.ds(vi * L, L)]
            ison = m > 0
            pos = jnp.full((L,), a + vi * L, jnp.int32) + lane
            plsc.store_compressed(ibuf.at[pl.ds(c, L)], pos, mask=ison)
            return c + jnp.sum(jnp.where(ison, 1, 0)), vi + 1

        lax.while_loop(wcond, wbody, (coll, jnp.int32(4)))

        vmax = jnp.full((L,), V - 1, jnp.int32)
        vmin = jnp.zeros((L,), jnp.int32)
        for t in range(IBUF // L):
            ibuf[pl.ds(t * L, L)] = jnp.clip(ibuf[pl.ds(t * L, L)], vmin,
                                             vmax)

    with jax.named_scope("gather"):
        obase = wid * BPW
        sems = (sem0, sem1)
        h = pltpu.async_copy(states_hbm.at[ibuf.at[pl.ds(0, GB)]],
                             rowbuf.at[0], sem0)
        for j in range(NGB):
            if j + 1 < NGB:
                h_next = pltpu.async_copy(
                    states_hbm.at[ibuf.at[pl.ds((j + 1) * GB, GB)]],
                    rowbuf.at[(j + 1) % 2], sems[(j + 1) % 2])
            h.wait()
            pltpu.sync_copy(rowbuf.at[j % 2],
                            out_hbm.at[pl.ds(obase + j * GB, GB)])
            if j + 1 < NGB:
                h = h_next


def kernel(inputs, states, importance_scores, active_mask):
    maskp = lax.bitcast_convert_type(
        active_mask.astype(jnp.uint8).reshape(PW, 4), jnp.int32)
    mask32 = jnp.pad(active_mask.astype(jnp.int32), (0, WWIN))
    return _compact_gather(maskp, mask32, states)


# speedup vs baseline: 1.2659x; 1.2659x over previous
import functools

import jax
import jax.numpy as jnp
from jax import lax
from jax.experimental import pallas as pl
from jax.experimental.pallas import tpu as pltpu
from jax.experimental.pallas import tpu_sc as plsc

V = 65536
D = 256
B = 49152
NC, NS, L = 2, 16, 16
NW = NC * NS
BPW = B // NW
PW = V // 4
NPV = PW // L
BLK = 16
NBLK = NPV // BLK
WWIN = 18432
A_C = V - WWIN
BCAP = A_C // (BLK * 4 * L)
GB = 96
NGB = BPW // GB
IBUF = BPW + 4 * L

_mesh = plsc.VectorSubcoreMesh(core_axis_name="c", subcore_axis_name="s")
_params = pltpu.CompilerParams(needs_layout_passes=False)


def _bytesum(v):
    s = v + (v >> 16)
    s = s + (s >> 8)
    return s & 0xFF


@functools.partial(
    pl.kernel,
    out_type=jax.ShapeDtypeStruct((B, D), jnp.float32),
    mesh=_mesh,
    scratch_types=[
        pltpu.VMEM((PW,), jnp.int32),
        pltpu.VMEM((NBLK, L), jnp.int32),
        pltpu.VMEM((IBUF,), jnp.int32),
        pltpu.VMEM((2, GB, D), jnp.float32),
        pltpu.SemaphoreType.DMA,
        pltpu.SemaphoreType.DMA,
    ],
    compiler_params=_params,
)
def _compact_gather(maskp_hbm, states_hbm, out_hbm, pbuf, bsums, ibuf,
                    rowbuf, sem0, sem1):
    wid = lax.axis_index("s") * NC + lax.axis_index("c")
    lane = lax.iota(jnp.int32, L)
    target = jnp.full((L,), wid * BPW, jnp.int32)

    with jax.named_scope("mask_load"):
        pltpu.sync_copy(maskp_hbm, pbuf)

    with jax.named_scope("locate"):
        zero = jnp.zeros((L,), jnp.int32)

        def s1body(b_, _):
            acc = zero
            for t in range(BLK):
                acc = acc + _bytesum(pbuf[pl.ds(b_ * (BLK * L) + t * L, L)])
            bsums[b_, pl.ds(0, L)] = acc
            return 0

        lax.fori_loop(0, NBLK, s1body, 0)

        acc, startb, accsel = zero, zero, zero
        pre_cap = zero
        for s in range(NBLK):
            if s == BCAP:
                pre_cap = acc
            tot = jnp.full((L,), jnp.sum(bsums[s, pl.ds(0, L)]), jnp.int32)
            take = (acc + tot) <= target
            startb = startb + jnp.where(take, 1, 0)
            accsel = jnp.where(take, acc + tot, accsel)
            acc = acc + tot
        bstar = jnp.max(startb)

        acc2, startv, accsel2 = zero, zero, zero
        for t in range(BLK):
            v = pbuf[pl.ds(bstar * (BLK * L) + t * L, L)]
            tot = jnp.full((L,), jnp.sum(_bytesum(v)), jnp.int32)
            take = (acc2 + tot) <= (target - accsel)
            startv = startv + jnp.where(take, 1, 0)
            accsel2 = jnp.where(take, acc2 + tot, accsel2)
            acc2 = acc2 + tot
        a = (bstar * BLK + jnp.max(startv)) * (4 * L)

    with jax.named_scope("walk"):
        a_spl = jnp.full((L,), a, jnp.int32)
        k0 = target - accsel - accsel2
        srem_spl = jnp.where(a_spl > A_C, target - pre_cap, k0)
        srem0 = jnp.max(srem_spl)
        pv0 = jnp.minimum(a, A_C) // (4 * L)

        def tot_of(vi):
            return jnp.sum(_bytesum(pbuf[pl.ds(vi * L, L)]))

        def pcond(c):
            srem, vi, tnext = c
            return (tnext <= srem) & (srem > 0) & (vi < NPV - 1)

        def pbody(c):
            srem, vi, tnext = c
            return srem - tnext, vi + 1, tot_of(vi + 1)

        srem1, vi1, _ = lax.while_loop(pcond, pbody,
                                       (srem0, pv0, tot_of(pv0)))

        v = pbuf[pl.ds(vi1 * L, L)]
        coll = jnp.int32(0)
        srs = jnp.full((L,), srem1, jnp.int32)
        for k in range(4):
            m = (v >> (8 * k)) & 1
            ison = m > 0
            pref = plsc.cumsum(m) - m
            keep = ison & (pref >= srs)
            pos = jnp.full((L,), vi1 * (4 * L) + k * L, jnp.int32) + lane
            plsc.store_compressed(ibuf.at[pl.ds(coll, L)], pos, mask=keep)
            coll = coll + jnp.sum(jnp.where(keep, 1, 0))
            srs = jnp.maximum(srs - plsc.all_reduce_population_count(ison),
                              0)

        def wcond(carry):
            c, vi = carry
            return (c < BPW) & (vi < NPV)

        def wbody(carry):
            c, vi = carry
            vv = pbuf[pl.ds(vi * L, L)]
            for k in range(4):
                m = (vv >> (8 * k)) & 1
                pos = jnp.full((L,), vi * (4 * L) + k * L, jnp.int32) + lane
                plsc.store_compressed(ibuf.at[pl.ds(c, L)], pos,
                                      mask=m > 0)
                c = c + jnp.sum(m)
            return c, vi + 1

        lax.while_loop(wcond, wbody, (coll, vi1 + 1))

        vmax = jnp.full((L,), V - 1, jnp.int32)
        vmin = jnp.zeros((L,), jnp.int32)
        for t in range(IBUF // L):
            ibuf[pl.ds(t * L, L)] = jnp.clip(ibuf[pl.ds(t * L, L)], vmin,
                                             vmax)

    with jax.named_scope("gather"):
        obase = wid * BPW
        sems = (sem0, sem1)
        h = pltpu.async_copy(states_hbm.at[ibuf.at[pl.ds(0, GB)]],
                             rowbuf.at[0], sem0)
        for j in range(NGB):
            if j + 1 < NGB:
                h_next = pltpu.async_copy(
                    states_hbm.at[ibuf.at[pl.ds((j + 1) * GB, GB)]],
                    rowbuf.at[(j + 1) % 2], sems[(j + 1) % 2])
            h.wait()
            pltpu.sync_copy(rowbuf.at[j % 2],
                            out_hbm.at[pl.ds(obase + j * GB, GB)])
            if j + 1 < NGB:
                h = h_next


def kernel(inputs, states, importance_scores, active_mask):
    maskp = lax.bitcast_convert_type(
        active_mask.astype(jnp.uint8).reshape(NPV, 4, L).transpose(0, 2, 1),
        jnp.int32).reshape(PW)
    return _compact_gather(maskp, states)
